# Initial kernel scaffold; baseline (speedup 1.0000x reference)
import jax
import jax.numpy as jnp
from jax.experimental import pallas as pl


def kernel(x_high, x_low, W_down_h, b_down_h, W_down_l, b_down_l, W_match, b_match, W_edge1, gamma, beta, W_edge2, W_getv, b_getv):
    raise NotImplementedError("write your pallas kernel here")


# speedup vs baseline: 1.8270x; 1.8270x over previous
import functools

import jax
import jax.numpy as jnp
import numpy as np
from jax import lax
from jax.experimental import pallas as pl
from jax.experimental.pallas import tpu as pltpu

N = 8
C = 192
DIME = 64
HH = 64
HL = 128
P = 128
NPIX_H = HH * HH
NPIX_L = HL * HL
EPS = 1e-5


def _resize_matrix(src, dst):
    ys = np.linspace(0.0, src - 1.0, dst).astype(np.float32)
    y0 = np.floor(ys).astype(np.int32)
    y1 = np.minimum(y0 + 1, src - 1)
    wy = (ys - y0).astype(np.float32)
    R = np.zeros((dst, src), np.float32)
    R[np.arange(dst), y0] += 1.0 - wy
    R[np.arange(dst), y1] += wy
    return R


def _bdot(a, b):
    return jnp.dot(a.astype(jnp.bfloat16), b.astype(jnp.bfloat16),
                   preferred_element_type=jnp.float32)


def _bdot_t(a, b):
    return lax.dot_general(a.astype(jnp.bfloat16), b.astype(jnp.bfloat16),
                           (((1,), (1,)), ((), ())),
                           preferred_element_type=jnp.float32)

def _hdot(a, b):
    return jnp.dot(a, b, preferred_element_type=jnp.float32,
                   precision=lax.Precision.HIGHEST)


def _sigmoid(x):
    return 1.0 / (1.0 + jnp.exp(-x))


def _k0_body(x_ref, W_ref, out_ref):
    out_ref[0] = _bdot(W_ref[...], x_ref[0])


CB = 48
NCB = C // CB


def _k1_body(xl_ref, xle_ref, xhe_ref, Wm_ref,
             bm_ref, G0T_ref, G1T_ref, wy_ref, omwy_ref,
             Pc_ref, o_ref, pool_ref):
    j = pl.program_id(1)
    xl = xl_ref[0]

    xl3 = xl.reshape(CB, HL, HL)
    rowsum = xl3.reshape(CB, HH, 2, HL).sum(axis=2)
    poolc = jnp.dot(rowsum.reshape(CB * HH, HL), Pc_ref[...],
                    preferred_element_type=jnp.float32, precision=lax.Precision.HIGHEST)
    pool_ref[0] = poolc.reshape(CB, HH, HH)

    @pl.when(j == NCB - 1)
    def _finish():
        xle = xle_ref[0]
        xhe = xhe_ref[0]
        wy = wy_ref[...]
        omwy = omwy_ref[...]
        xle3 = xle.reshape(DIME, HL, HL)
        T = xle3.transpose(0, 2, 1)
        Tf = T.reshape(DIME * HL, HL)
        topT = _hdot(Tf, G0T_ref[...]).reshape(DIME, HL, HH)
        botT = _hdot(Tf, G1T_ref[...]).reshape(DIME, HL, HH)
        vT = topT * omwy.reshape(1, 1, HH) + botT * wy.reshape(1, 1, HH)
        v = vT.transpose(0, 2, 1)
        vf = v.reshape(DIME * HH, HL)
        lg = _hdot(vf, G0T_ref[...]).reshape(DIME, HH, HH)
        rg = _hdot(vf, G1T_ref[...]).reshape(DIME, HH, HH)
        xle_r3 = lg * omwy.reshape(1, 1, HH) + rg * wy.reshape(1, 1, HH)
        xle_r = xle_r3.reshape(DIME, NPIX_H)

        cin = jnp.concatenate([xhe, xle_r], axis=0)
        p9 = _bdot(Wm_ref[...], cin)
        p93 = p9.reshape(9, HH, HH)
        o = jnp.zeros((HH, HH), jnp.float32)
        for dy in range(3):
            for dx in range(3):
                t = dy * 3 + dx
                mp = jnp.pad(p93[t], ((1, 1), (1, 1)))
                o = o + mp[dy:dy + HH, dx:dx + HH]
        o_ref[0] = o + bm_ref[0, 0]


def _k2_body(xh_ref, cert_ref, We1_ref, Q01_ref, Q01T_ref,
             G80_ref, G81_ref, G80T_ref, G81T_ref, wy8c_ref, omwy8c_ref,
             wx8r_ref, omwx8r_ref, h_ref, st_ref):
    cert = cert_ref[0]
    avg8 = _hdot(_hdot(Q01_ref[...], cert), Q01T_ref[...]) * (1.0 / 64.0)
    top8 = _hdot(G80_ref[...], avg8)
    bot8 = _hdot(G81_ref[...], avg8)
    v8 = top8 * omwy8c_ref[...] + bot8 * wy8c_ref[...]
    l8 = _hdot(v8, G80T_ref[...])
    r8 = _hdot(v8, G81T_ref[...])
    avg = l8 * omwx8r_ref[...] + r8 * wx8r_ref[...]
    xh3 = xh_ref[0].reshape(C, HH, HH)
    xhe3 = xh3 - xh3 * avg[None, :, :]
    hp = jnp.pad(xhe3, ((0, 0), (1, 1), (1, 1)))
    acc = jnp.zeros((C, NPIX_H), jnp.float32)
    for dy in range(3):
        for dx in range(3):
            t = dy * 3 + dx
            sh = hp[:, dy:dy + HH, dx:dx + HH].reshape(C, NPIX_H)
            acc = acc + _bdot(We1_ref[t], sh)
    h_ref[0] = acc
    s1 = acc.sum(axis=1, keepdims=True)
    s2 = (acc * acc).sum(axis=1, keepdims=True)
    st_ref[0] = jnp.concatenate([s1, s2], axis=1)


INV_CNT = 1.0 / float(N * NPIX_H)


def _k3a_body(h_ref, st_ref, vp_ref):
    mean = st_ref[...].sum(axis=0)[:, 0:1] * INV_CNT
    d = h_ref[0] - mean
    vp_ref[0] = (d * d).sum(axis=1, keepdims=True)


def _k3b_body(h_ref, st_ref, vp_ref, gam_ref, bet_ref, We2_ref, ep_ref):
    mean = st_ref[...].sum(axis=0)[:, 0:1] * INV_CNT
    var = vp_ref[...].sum(axis=0) * INV_CNT
    s = jnp.sqrt(var + EPS)
    hn = jnp.maximum(((h_ref[0] - mean) / s) * gam_ref[...] + bet_ref[...], 0.0)
    p9 = _bdot(We2_ref[...], hn)
    p93 = p9.reshape(9, HH, HH)
    o = jnp.zeros((HH, HH), jnp.float32)
    for dy in range(3):
        for dx in range(3):
            t = dy * 3 + dx
            mp = jnp.pad(p93[t], ((1, 1), (1, 1)))
            o = o + mp[dy:dy + HH, dx:dx + HH]
    ep_ref[0, 0] = o


def _k4_body(ep_ref, idx_ref):
    vals0 = ep_ref[...]
    iota = lax.broadcasted_iota(jnp.int32, (N, NPIX_H), 1)
    col = lax.broadcasted_iota(jnp.int32, (N, P), 1)
    neg = jnp.float32(-jnp.inf)

    def step(t, carry):
        vals, out = carry
        m = jnp.max(vals, axis=1, keepdims=True)
        cand = jnp.where(vals == m, iota, NPIX_H)
        sel = jnp.min(cand, axis=1, keepdims=True)
        out = jnp.where(col == t, sel, out)
        vals = jnp.where(iota == sel, neg, vals)
        return vals, out

    _, out = lax.fori_loop(0, P, step, (vals0, jnp.zeros((N, P), jnp.int32)))
    idx_ref[...] = out


def _k5_body(xh_ref, pool_ref, idx_ref, Wg_ref, bg_ref,
             hef_ref, lef_ref, prob_ref, lepe_ref):
    idx = idx_ref[0]
    oh = (lax.broadcasted_iota(jnp.int32, (NPIX_H, P), 0) == idx
          ).astype(jnp.float32)
    hef = jnp.dot(xh_ref[0], oh, preferred_element_type=jnp.float32, precision=lax.Precision.HIGHEST)
    lef = jnp.dot(pool_ref[0].reshape(C, NPIX_H), oh,
                  preferred_element_type=jnp.float32, precision=lax.Precision.HIGHEST)
    hef_ref[0] = hef
    lef_ref[0] = lef
    logits = _bdot_t(lef, hef)
    m = jnp.max(logits, axis=1, keepdims=True)
    e = jnp.exp(logits - m)
    prob_ref[0] = e / e.sum(axis=1, keepdims=True)
    hp = jnp.pad(hef, ((0, 0), (1, 1)))
    lepe = jnp.broadcast_to(bg_ref[...], (C, P))
    for t in range(3):
        lepe = lepe + _bdot(Wg_ref[t], hp[:, t:t + P])
    lepe_ref[0] = lepe


def _k6_body(prob_ref, v2_ref, lepe_ref, out_ref):
    out_ref[0] = _bdot(prob_ref[0], v2_ref[0]) + lepe_ref[0]


BLK_L = 2048


def _k7_body(xl_ref, outs_ref, lef_ref, idx_ref, fin_ref):
    j = pl.program_id(1)
    idx = idx_ref[0]
    lei = 2 * (idx % HH) + 256 * (idx // HH)
    local = lei - j * BLK_L
    onehot = (lax.broadcasted_iota(jnp.int32, (BLK_L, P), 0) == local
              ).astype(jnp.float32)
    fusion = outs_ref[0] + lef_ref[0]
    scat = lax.dot_general(fusion, onehot, (((1,), (1,)), ((), ())),
                           preferred_element_type=jnp.float32, precision=lax.Precision.HIGHEST)
    mask = lax.dot_general(jnp.ones((1, P), jnp.float32), onehot,
                           (((1,), (1,)), ((), ())),
                           preferred_element_type=jnp.float32, precision=lax.Precision.HIGHEST)
    fin_ref[0] = xl_ref[0] * (1.0 - mask) + scat


def kernel(x_high, x_low, W_down_h, b_down_h, W_down_l, b_down_l, W_match,
           b_match, W_edge1, gamma, beta, W_edge2, W_getv, b_getv):
    f32 = jnp.float32
    xh = x_high.reshape(N, C, NPIX_H)
    xl = x_low.reshape(N, C, NPIX_L)

    Wdh = W_down_h.reshape(DIME, C)
    Wdl = W_down_l.reshape(DIME, C)
    bdh = b_down_h.reshape(DIME, 1)
    bdl = b_down_l.reshape(DIME, 1)
    Wm = W_match.reshape(2 * DIME, 9).T
    bm = b_match.reshape(1, 1)
    We1 = W_edge1.transpose(2, 3, 0, 1).reshape(9, C, C)
    We2 = W_edge2.reshape(C, 9).T
    Wg = W_getv[:, :, 1, :].transpose(2, 0, 1)
    bg = b_getv.reshape(C, 1)
    gam = gamma.reshape(C, 1)
    bet = beta.reshape(C, 1)

    y0 = np.floor(np.linspace(0.0, HL - 1.0, HH)).astype(np.int64)
    y1 = np.minimum(y0 + 1, HL - 1)
    G0T = np.zeros((HL, HH), np.float32); G0T[y0, np.arange(HH)] = 1.0
    G1T = np.zeros((HL, HH), np.float32); G1T[y1, np.arange(HH)] = 1.0
    G0T, G1T = jnp.asarray(G0T), jnp.asarray(G1T)
    ys = jnp.linspace(0.0, HL - 1.0, HH, dtype=jnp.float32)
    wy = (ys - jnp.floor(ys)).reshape(1, HH)
    omwy = 1.0 - wy
    Q01 = np.zeros((8, HH), np.float32)
    for i in range(8):
        Q01[i, 8 * i:8 * i + 8] = 1.0
    Q01 = jnp.asarray(Q01)
    y08 = np.floor(np.linspace(0.0, 7.0, HH)).astype(np.int64)
    y18 = np.minimum(y08 + 1, 7)
    G80 = np.zeros((HH, 8), np.float32); G80[np.arange(HH), y08] = 1.0
    G81 = np.zeros((HH, 8), np.float32); G81[np.arange(HH), y18] = 1.0
    G80, G81 = jnp.asarray(G80), jnp.asarray(G81)
    ys8 = jnp.linspace(0.0, 7.0, HH, dtype=jnp.float32)
    wy8 = ys8 - jnp.floor(ys8)
    wy8c = wy8.reshape(HH, 1)
    omwy8c = 1.0 - wy8c
    wx8r = wy8.reshape(1, HH)
    omwx8r = 1.0 - wx8r
    Pc = np.zeros((HL, HH), np.float32)
    for w in range(HL):
        Pc[w, w // 2] = 0.25
    Pc = jnp.asarray(Pc)

    full = lambda shape: pl.BlockSpec(shape, lambda n: (0,) * len(shape))
    per_n = lambda shape: pl.BlockSpec((1,) + shape, lambda n: (n,) + (0,) * len(shape))

    full2 = lambda shape: pl.BlockSpec(shape, lambda n, j: (0,) * len(shape))
    xle = pl.pallas_call(
        _k0_body,
        grid=(N, 4),
        in_specs=[pl.BlockSpec((1, C, NPIX_H), lambda n, j: (n, 0, j)),
                  full2((DIME, C))],
        out_specs=pl.BlockSpec((1, DIME, NPIX_H), lambda n, j: (n, 0, j)),
        out_shape=jax.ShapeDtypeStruct((N, DIME, NPIX_L), f32),
    )(xl, Wdl)
    xhe = pl.pallas_call(
        _k0_body,
        grid=(N, 1),
        in_specs=[pl.BlockSpec((1, C, NPIX_H), lambda n, j: (n, 0, 0)),
                  full2((DIME, C))],
        out_specs=pl.BlockSpec((1, DIME, NPIX_H), lambda n, j: (n, 0, 0)),
        out_shape=jax.ShapeDtypeStruct((N, DIME, NPIX_H), f32),
    )(xh, Wdh)

    o, pool = pl.pallas_call(
        _k1_body,
        grid=(N, NCB),
        in_specs=[pl.BlockSpec((1, CB, NPIX_L), lambda n, j: (n, j, 0)),
                  pl.BlockSpec((1, DIME, NPIX_L), lambda n, j: (n, 0, 0)),
                  pl.BlockSpec((1, DIME, NPIX_H), lambda n, j: (n, 0, 0)),
                  full2((9, 2 * DIME)), full2((1, 1)),
                  full2((HL, HH)), full2((HL, HH)), full2((1, HH)),
                  full2((1, HH)), full2((HL, HH))],
        out_specs=[pl.BlockSpec((1, HH, HH), lambda n, j: (n, 0, 0)),
                   pl.BlockSpec((1, CB, HH, HH), lambda n, j: (n, j, 0, 0))],
        out_shape=[jax.ShapeDtypeStruct((N, HH, HH), f32),
                   jax.ShapeDtypeStruct((N, C, HH, HH), f32)],
    )(xl, xle, xhe, Wm, bm, G0T, G1T, wy, omwy, Pc)

    cert = jax.nn.sigmoid(o)

    h, st = pl.pallas_call(
        _k2_body,
        grid=(N,),
        in_specs=[per_n((C, NPIX_H)), per_n((HH, HH)), full((9, C, C)),
                  full((8, HH)), full((HH, 8)), full((HH, 8)), full((HH, 8)),
                  full((8, HH)), full((8, HH)), full((HH, 1)), full((HH, 1)),
                  full((1, HH)), full((1, HH))],
        out_specs=[per_n((C, NPIX_H)), per_n((C, 2))],
        out_shape=[jax.ShapeDtypeStruct((N, C, NPIX_H), f32),
                   jax.ShapeDtypeStruct((N, C, 2), f32)],
    )(xh, cert, We1, Q01, Q01.T, G80, G81, G80.T, G81.T,
      wy8c, omwy8c, wx8r, omwx8r)

    vp = pl.pallas_call(
        _k3a_body,
        grid=(N,),
        in_specs=[per_n((C, NPIX_H)), full((N, C, 2))],
        out_specs=per_n((C, 1)),
        out_shape=jax.ShapeDtypeStruct((N, C, 1), f32),
    )(h, st)

    edge_pred = pl.pallas_call(
        _k3b_body,
        grid=(N,),
        in_specs=[per_n((C, NPIX_H)), full((N, C, 2)), full((N, C, 1)),
                  full((C, 1)), full((C, 1)), full((9, C))],
        out_specs=per_n((1, HH, HH)),
        out_shape=jax.ShapeDtypeStruct((N, 1, HH, HH), f32),
    )(h, st, vp, gam, bet, We2)

    point_indices = pl.pallas_call(
        _k4_body,
        in_specs=[pl.BlockSpec((N, NPIX_H), lambda: (0, 0))],
        out_specs=pl.BlockSpec((N, P), lambda: (0, 0)),
        out_shape=jax.ShapeDtypeStruct((N, P), jnp.int32),
    )(edge_pred.reshape(N, NPIX_H))

    idx3 = point_indices.reshape(N, 1, P)

    hef, lef, prob, lepe = pl.pallas_call(
        _k5_body,
        grid=(N,),
        in_specs=[per_n((C, NPIX_H)), per_n((C, HH, HH)), per_n((1, P)),
                  full((3, C, C)), full((C, 1))],
        out_specs=[per_n((C, P)), per_n((C, P)), per_n((C, C)),
                   per_n((C, P))],
        out_shape=[jax.ShapeDtypeStruct((N, C, P), f32),
                   jax.ShapeDtypeStruct((N, C, P), f32),
                   jax.ShapeDtypeStruct((N, C, C), f32),
                   jax.ShapeDtypeStruct((N, C, P), f32)],
    )(xh, pool, idx3, Wg, bg)

    v2 = hef.reshape(N, P, C).transpose(0, 2, 1)
    lepe_s = lepe.reshape(N, P, C).transpose(0, 2, 1)

    attn_out = pl.pallas_call(
        _k6_body,
        grid=(N,),
        in_specs=[per_n((C, C)), per_n((C, P)), per_n((C, P))],
        out_specs=per_n((C, P)),
        out_shape=jax.ShapeDtypeStruct((N, C, P), f32),
    )(prob, v2, lepe_s)

    out_s = attn_out.transpose(0, 2, 1).reshape(N, C, P)

    nblk = NPIX_L // BLK_L
    final = pl.pallas_call(
        _k7_body,
        grid=(N, nblk),
        in_specs=[pl.BlockSpec((1, C, BLK_L), lambda n, j: (n, 0, j)),
                  pl.BlockSpec((1, C, P), lambda n, j: (n, 0, 0)),
                  pl.BlockSpec((1, C, P), lambda n, j: (n, 0, 0)),
                  pl.BlockSpec((1, 1, P), lambda n, j: (n, 0, 0))],
        out_specs=pl.BlockSpec((1, C, BLK_L), lambda n, j: (n, 0, j)),
        out_shape=jax.ShapeDtypeStruct((N, C, NPIX_L), f32),
    )(xl, out_s, lef, idx3)

    return (final.reshape(N, C, HL, HL), edge_pred)
